# Initial kernel scaffold; baseline (speedup 1.0000x reference)
import jax
import jax.numpy as jnp
from jax.experimental import pallas as pl


def kernel(features, edge_index, W1, b1, W2, b2):
    raise NotImplementedError("write your pallas kernel here")


# speedup vs baseline: 15.4690x; 15.4690x over previous
import functools

import jax
import jax.numpy as jnp
from jax import lax
from jax.experimental import pallas as pl
from jax.experimental.pallas import tpu as pltpu
from jax.experimental.pallas import tpu_sc as plsc

N = 10000
E = 80000
D_IN = 1433
D_H = 16
D_OUT = 7

NC = 2
NS = 16
NW = NC * NS

CH = 128
EPW = 2560
NCHUNK = EPW // CH
E_PAD = EPW * NW
AN = 10240
STRIPE = AN // NS
TRASH = N

MBLK = 1000
GRID = N // MBLK


def _mm_body(x_ref, w_ref, o_ref):
    y = jnp.dot(x_ref[...], w_ref[...], preferred_element_type=jnp.float32)
    col = lax.broadcasted_iota(jnp.int32, (MBLK, 2 * D_H), 1)
    o_ref[...] = y + jnp.where(col == D_H, 1.0, 0.0).astype(jnp.float32)


def _project(features, w1p):
    return pl.pallas_call(
        _mm_body,
        grid=(GRID,),
        in_specs=[
            pl.BlockSpec((MBLK, D_IN), lambda i: (i, 0)),
            pl.BlockSpec((D_IN, 2 * D_H), lambda i: (0, 0)),
        ],
        out_specs=pl.BlockSpec((MBLK, 2 * D_H), lambda i: (i, 0)),
        out_shape=jax.ShapeDtypeStruct((N, 2 * D_H), jnp.float32),
    )(features, w1p)


def _make_segsum(d):
    mesh = plsc.VectorSubcoreMesh(core_axis_name="c", subcore_axis_name="s")

    @functools.partial(
        pl.kernel,
        out_type=jax.ShapeDtypeStruct((NC * AN, d), jnp.float32),
        mesh=mesh,
        compiler_params=pltpu.CompilerParams(use_tc_tiling_on_sc=False),
        scratch_types=[
            pltpu.VMEM((NCHUNK, CH), jnp.int32),
            pltpu.VMEM((NCHUNK, CH), jnp.int32),
            pltpu.VMEM((EPW, d), jnp.float32),
            pltpu.VMEM_SHARED((AN, d), jnp.float32),
            pltpu.SemaphoreType.DMA,
        ],
    )
    def segsum(y_hbm, src_hbm, dst_hbm, zer_hbm, out_hbm,
               src_v, dst_v, rows_v, acc, sem):
        c = lax.axis_index("c")
        s = lax.axis_index("s")
        wid = s * NC + c

        pltpu.sync_copy(zer_hbm.at[pl.ds(s * STRIPE, STRIPE)],
                        acc.at[pl.ds(s * STRIPE, STRIPE)])

        pltpu.sync_copy(src_hbm.at[wid], src_v)
        pltpu.sync_copy(dst_hbm.at[wid], dst_v)

        cps = []
        for j in range(NCHUNK):
            cps.append(pltpu.async_copy(
                y_hbm.at[src_v.at[j]],
                rows_v.at[pl.ds(j * CH, CH)], sem))
        for cp in cps:
            cp.wait()

        plsc.subcore_barrier()

        for j in range(NCHUNK):
            pltpu.sync_copy(rows_v.at[pl.ds(j * CH, CH)],
                            acc.at[dst_v.at[j]], add=True)

        plsc.subcore_barrier()

        pltpu.sync_copy(acc.at[pl.ds(s * STRIPE, STRIPE)],
                        out_hbm.at[pl.ds(c * AN + s * STRIPE, STRIPE)])

    return segsum


_segsum_32 = _make_segsum(2 * D_H)
_segsum_16 = _make_segsum(D_H)


def _l1_body(a0_ref, a1_ref, y_ref, b_ref, h_ref, d_ref):
    a = a0_ref[0] + a1_ref[0]
    ssum = a[:, :D_H]
    deg = a[:, D_H:D_H + 1]
    dinv = 1.0 / (deg + 1.0)
    h = (ssum + y_ref[:, :D_H]) * dinv + b_ref[...]
    h_ref[...] = jnp.maximum(h, 0.0)
    d_ref[...] = jnp.broadcast_to(dinv, (MBLK, D_H))


def _layer1_post(acc1, y1p, b1r):
    return pl.pallas_call(
        _l1_body,
        grid=(GRID,),
        in_specs=[
            pl.BlockSpec((1, MBLK, 2 * D_H), lambda i: (0, i, 0)),
            pl.BlockSpec((1, MBLK, 2 * D_H), lambda i: (1, i, 0)),
            pl.BlockSpec((MBLK, 2 * D_H), lambda i: (i, 0)),
            pl.BlockSpec((1, D_H), lambda i: (0, 0)),
        ],
        out_specs=[
            pl.BlockSpec((MBLK, D_H), lambda i: (i, 0)),
            pl.BlockSpec((MBLK, D_H), lambda i: (i, 0)),
        ],
        out_shape=[
            jax.ShapeDtypeStruct((N, D_H), jnp.float32),
            jax.ShapeDtypeStruct((N, D_H), jnp.float32),
        ],
    )(acc1, acc1, y1p, b1r)


def _l2_body(a0_ref, a1_ref, h_ref, d_ref, w_ref, b_ref, o_ref):
    st = a0_ref[0] + a1_ref[0]
    z = (st + h_ref[...]) * d_ref[...]
    logits = jnp.dot(z, w_ref[...], preferred_element_type=jnp.float32)
    logits = logits + b_ref[...]
    col = lax.broadcasted_iota(jnp.int32, (MBLK, 128), 1)
    x = jnp.where(col < D_OUT, logits, -1e30)
    m = jnp.max(x, axis=1, keepdims=True)
    ex = jnp.exp(x - m)
    ssum = jnp.sum(ex, axis=1, keepdims=True)
    o_ref[...] = (x - m) - jnp.log(ssum)


def _layer2(acc2, h1, dinv, w2p, b2p):
    return pl.pallas_call(
        _l2_body,
        grid=(GRID,),
        in_specs=[
            pl.BlockSpec((1, MBLK, D_H), lambda i: (0, i, 0)),
            pl.BlockSpec((1, MBLK, D_H), lambda i: (1, i, 0)),
            pl.BlockSpec((MBLK, D_H), lambda i: (i, 0)),
            pl.BlockSpec((MBLK, D_H), lambda i: (i, 0)),
            pl.BlockSpec((D_H, 128), lambda i: (0, 0)),
            pl.BlockSpec((1, 128), lambda i: (0, 0)),
        ],
        out_specs=pl.BlockSpec((MBLK, 128), lambda i: (i, 0)),
        out_shape=jax.ShapeDtypeStruct((N, 128), jnp.float32),
    )(acc2, acc2, h1, dinv, w2p, b2p)


def kernel(features, edge_index, W1, b1, W2, b2):
    src = edge_index[0].astype(jnp.int32)
    dst = edge_index[1].astype(jnp.int32)
    srcp = jnp.concatenate(
        [src, jnp.zeros((E_PAD - E,), jnp.int32)]).reshape(NW, NCHUNK, CH)
    dstp = jnp.concatenate(
        [dst, jnp.full((E_PAD - E,), TRASH, jnp.int32)]).reshape(NW, NCHUNK, CH)

    w1p = jnp.pad(W1, ((0, 0), (0, D_H)))
    b1r = b1.reshape(1, D_H)
    w2p = jnp.pad(W2, ((0, 0), (0, 128 - D_OUT)))
    b2p = jnp.pad(b2, (0, 128 - D_OUT)).reshape(1, 128)
    zer32 = jnp.zeros((AN, 2 * D_H), jnp.float32)
    zer16 = jnp.zeros((AN, D_H), jnp.float32)

    y1p = _project(features, w1p)

    acc1 = _segsum_32(y1p, srcp, dstp, zer32).reshape(NC, AN, 2 * D_H)
    h1, dinv = _layer1_post(acc1, y1p, b1r)

    acc2 = _segsum_16(h1, srcp, dstp, zer16).reshape(NC, AN, D_H)
    out = _layer2(acc2, h1, dinv, w2p, b2p)
    return out[:, :D_OUT]
